# Initial kernel scaffold; baseline (speedup 1.0000x reference)
import jax
import jax.numpy as jnp
from jax.experimental import pallas as pl


def kernel(inputs, token_table, pos_table):
    raise NotImplementedError("write your pallas kernel here")


# speedup vs baseline: 1.8405x; 1.8405x over previous
import functools

import jax
import jax.numpy as jnp
from jax import lax
from jax.experimental import pallas as pl
from jax.experimental.pallas import tpu as pltpu
from jax.experimental.pallas import tpu_sc as plsc

LANES = 16
NUM_CORES = 2
NUM_SUBCORES = 16
NUM_WORKERS = NUM_CORES * NUM_SUBCORES
CHUNK = 128


def _sc_embed(idx_flat, token_table, pos_table):
    b_total = idx_flat.shape[0]
    maxlen, embed = pos_table.shape
    b_per_w = b_total // NUM_WORKERS
    n_chunks = b_per_w // CHUNK

    mesh = plsc.VectorSubcoreMesh(core_axis_name="c", subcore_axis_name="s")

    @functools.partial(
        pl.kernel,
        mesh=mesh,
        out_type=jax.ShapeDtypeStruct((b_total, embed), jnp.float32),
        scratch_types=[
            pltpu.VMEM((CHUNK,), jnp.int32),
            pltpu.VMEM((CHUNK, embed), jnp.float32),
            pltpu.VMEM((2 * maxlen, embed), jnp.float32),
            pltpu.SemaphoreType.DMA,
        ],
    )
    def k(table_hbm, idx_hbm, pos_hbm, out_hbm, idx_v, rows_v, pos_v, sem):
        wid = lax.axis_index("s") * NUM_CORES + lax.axis_index("c")
        base_w = wid * b_per_w

        pltpu.sync_copy(pos_hbm, pos_v.at[pl.ds(0, maxlen)])
        pltpu.sync_copy(pos_hbm, pos_v.at[pl.ds(maxlen, maxlen)])

        def chunk_body(g, carry):
            base = base_w + g * CHUNK
            p0 = lax.rem(g * CHUNK, maxlen)
            pltpu.sync_copy(idx_hbm.at[pl.ds(base, CHUNK)], idx_v)
            pltpu.async_copy(table_hbm.at[idx_v], rows_v, sem).wait()

            def row_body(i, c2):
                pr = p0 + i
                for j in range(embed // LANES):
                    sl = pl.ds(j * LANES, LANES)
                    rows_v[i, sl] = rows_v[i, sl] + pos_v[pr, sl]
                return c2

            lax.fori_loop(0, CHUNK, row_body, 0)
            pltpu.sync_copy(rows_v, out_hbm.at[pl.ds(base, CHUNK)])
            return carry

        lax.fori_loop(0, n_chunks, chunk_body, 0)

    return k(token_table, idx_flat, pos_table)


def kernel(inputs, token_table, pos_table):
    batch, maxlen = inputs.shape
    embed = token_table.shape[1]
    idx_flat = inputs.reshape(-1).astype(jnp.int32)
    out = _sc_embed(idx_flat, token_table, pos_table)
    return out.reshape(batch, maxlen, embed)


# speedup vs baseline: 9.0234x; 4.9027x over previous
import functools

import jax
import jax.numpy as jnp
from jax import lax
from jax.experimental import pallas as pl
from jax.experimental.pallas import tpu as pltpu
from jax.experimental.pallas import tpu_sc as plsc

LANES = 16
NUM_CORES = 2
NUM_SUBCORES = 16
NUM_WORKERS = NUM_CORES * NUM_SUBCORES
G0 = 128


def _sc_embed(idx_flat, token_table, pos_table):
    b_total = idx_flat.shape[0]
    maxlen, embed = pos_table.shape
    chunk = maxlen
    b_per_w = b_total // NUM_WORKERS
    n_chunks = b_per_w // chunk

    mesh = plsc.VectorSubcoreMesh(core_axis_name="c", subcore_axis_name="s")

    @functools.partial(
        pl.kernel,
        mesh=mesh,
        out_type=jax.ShapeDtypeStruct((b_total, embed), jnp.float32),
        scratch_types=[
            pltpu.VMEM((b_per_w,), jnp.int32),
            pltpu.VMEM((chunk, embed), jnp.float32),
            pltpu.VMEM((chunk, embed), jnp.float32),
            pltpu.VMEM((chunk, embed), jnp.float32),
            pltpu.VMEM((chunk, embed), jnp.float32),
            pltpu.SemaphoreType.DMA,
            pltpu.SemaphoreType.DMA,
            pltpu.SemaphoreType.DMA,
            pltpu.SemaphoreType.DMA,
            pltpu.SemaphoreType.DMA,
            pltpu.SemaphoreType.DMA,
        ],
    )
    def k(table_hbm, idx_hbm, pos_hbm, out_hbm,
          idx_v, pos_v, rows0, rows1, rows2,
          g0, g1, g2, o0, o1, o2):
        rows = (rows0, rows1, rows2)
        gsem = (g0, g1, g2)
        osem = (o0, o1, o2)

        wid = lax.axis_index("s") * NUM_CORES + lax.axis_index("c")
        base_w = wid * b_per_w

        pltpu.sync_copy(idx_hbm.at[pl.ds(base_w, b_per_w)], idx_v)
        pltpu.sync_copy(pos_hbm, pos_v)

        def fire_gather(g, b):
            off = g * chunk
            pltpu.async_copy(
                table_hbm.at[idx_v.at[pl.ds(off, G0)]],
                rows[b].at[pl.ds(0, G0)], gsem[b])
            pltpu.async_copy(
                table_hbm.at[idx_v.at[pl.ds(off + G0, chunk - G0)]],
                rows[b].at[pl.ds(G0, chunk - G0)], gsem[b])

        def wait_gather(g, b):
            off = g * chunk
            pltpu.make_async_copy(
                table_hbm.at[idx_v.at[pl.ds(off, G0)]],
                rows[b].at[pl.ds(0, G0)], gsem[b]).wait()
            pltpu.make_async_copy(
                table_hbm.at[idx_v.at[pl.ds(off + G0, chunk - G0)]],
                rows[b].at[pl.ds(G0, chunk - G0)], gsem[b]).wait()

        def add_pos(b):
            r = rows[b]

            @plsc.parallel_loop(0, chunk, step=1, unroll=2)
            def _(i):
                for j in range(embed // LANES):
                    sl = pl.ds(j * LANES, LANES)
                    plsc.addupdate(r.at[i, sl], pos_v[i, sl])

        def fire_out(g, b):
            pltpu.async_copy(
                rows[b], out_hbm.at[pl.ds(base_w + g * chunk, chunk)], osem[b])

        def wait_out_prev(g, b):
            bp = (b + 2) % 3
            gp = g - 1
            pltpu.make_async_copy(
                rows[bp], out_hbm.at[pl.ds(base_w + gp * chunk, chunk)],
                osem[bp]).wait()

        def body(g, b, fire_next):
            wait_gather(g, b)
            add_pos(b)
            fire_out(g, b)
            if fire_next:
                wait_out_prev(g, b)
                fire_gather(g + 2, (b + 2) % 3)

        fire_gather(0, 0)
        fire_gather(1, 1)

        wait_gather(0, 0)
        add_pos(0)
        fire_out(0, 0)
        fire_gather(2, 2)

        def h_body(h, carry):
            gb = 3 * h + 1
            body(gb, 1, True)
            body(gb + 1, 2, True)
            body(gb + 2, 0, True)
            return carry

        lax.fori_loop(0, (n_chunks - 5) // 3, h_body, 0)

        body(n_chunks - 4, (n_chunks - 4) % 3, True)
        body(n_chunks - 3, (n_chunks - 3) % 3, True)
        for g in (n_chunks - 2, n_chunks - 1):
            b = g % 3
            wait_gather(g, b)
            add_pos(b)
            fire_out(g, b)
            wait_out_prev(g, b)
        bl = (n_chunks - 1) % 3
        pltpu.make_async_copy(
            rows[bl],
            out_hbm.at[pl.ds(base_w + (n_chunks - 1) * chunk, chunk)],
            osem[bl]).wait()

    return k(token_table, idx_flat, pos_table)


def kernel(inputs, token_table, pos_table):
    batch, maxlen = inputs.shape
    embed = token_table.shape[1]
    idx_flat = inputs.reshape(-1).astype(jnp.int32)
    out = _sc_embed(idx_flat, token_table, pos_table)
    return out.reshape(batch, maxlen, embed)
